# baseline (device time: 87231 ns/iter reference)
import functools

import jax
import jax.numpy as jnp
from jax import lax
from jax.experimental import pallas as pl
from jax.experimental.pallas import tpu as pltpu

QROWS = 1024

S0, S1, S2 = 384, 320, 320
PIECE_OFF = (0, S0, S0 + S1)
CHUNK_ROWS = ((64, 128, 192), (160, 160), (160, 160))
YTAIL_ROWS = (192, 192)


def _chunks(pieces):
    out = []
    for q_rel, j, chunk_list in pieces:
        off = PIECE_OFF[j]
        for r in chunk_list:
            out.append((q_rel, off, r))
            off += r
    return out


C0, C1, C2 = CHUNK_ROWS
Y_STREAM = _chunks([(0, 0, C0), (0, 1, C1), (0, 2, C2), (2, 0, YTAIL_ROWS)])
CW_OUT = _chunks([(0, 0, C0), (0, 1, C1), (0, 2, C2), (3, 2, C2)])
CCW_OUT = _chunks([(0, 0, C0), (0, 1, C1), (0, 2, C2), (1, 1, C1)])
CW_IN = _chunks([(3, 0, C0), (3, 1, C1), (3, 2, C2), (2, 2, C2)])
CCW_IN = _chunks([(1, 0, C0), (1, 1, C1), (1, 2, C2), (2, 1, C1)])

MESH = pltpu.DeviceIdType.MESH


def kernel(x):
    m, n = x.shape

    def body(x_ref, out_ref, comm_ref,
             y_s, y_r, cw_s, cw_r, ccw_s, ccw_r):
        my_x = lax.axis_index("x")
        my_y = lax.axis_index("y")
        my_z = lax.axis_index("z")
        k = 2 * my_x + (my_x ^ my_z)
        even = (k % 2) == 0
        nxt = (jnp.where(even, my_x, 1 - my_x), my_y,
               jnp.where(even, 1 - my_z, my_z))
        prv = (jnp.where(even, 1 - my_x, my_x), my_y,
               jnp.where(even, my_z, 1 - my_z))
        peer_y = (my_x, 1 - my_y, my_z)

        def rows(q_rel, off, nrows):
            return pl.ds(((k + q_rel) % 4) * QROWS + off, nrows)

        barrier_sem = pltpu.get_barrier_semaphore()
        for p in (peer_y, nxt, prv):
            pl.semaphore_signal(barrier_sem, inc=1, device_id=p,
                                device_id_type=MESH)
        pl.semaphore_wait(barrier_sem, 3)

        y_rd = []
        for c, (q_rel, off, nr) in enumerate(Y_STREAM):
            sl = rows(q_rel, off, nr)
            r = pltpu.make_async_remote_copy(
                src_ref=x_ref.at[sl, :], dst_ref=comm_ref.at[sl, :],
                send_sem=y_s.at[c], recv_sem=y_r.at[c],
                device_id=peer_y, device_id_type=MESH)
            r.start()
            y_rd.append(r)

        cw_rd = [None] * len(CW_OUT)
        ccw_rd = [None] * len(CCW_OUT)

        def ring_start(stream, sems_s, sems_r, target, c, lst):
            q_rel, off, nr = stream[c]
            sl = rows(q_rel, off, nr)
            r = pltpu.make_async_remote_copy(
                src_ref=out_ref.at[sl, :], dst_ref=out_ref.at[sl, :],
                send_sem=sems_s.at[c], recv_sem=sems_r.at[c],
                device_id=target, device_id_type=MESH)
            r.start()
            lst[c] = r

        def in_wait(stream, sems_r, c):
            q_rel, off, nr = stream[c]
            sl = rows(q_rel, off, nr)
            r = pltpu.make_async_remote_copy(
                src_ref=out_ref.at[sl, :], dst_ref=out_ref.at[sl, :],
                send_sem=y_s.at[0], recv_sem=sems_r.at[c],
                device_id=peer_y, device_id_type=MESH)
            r.wait_recv()

        def add(stream, c):
            q_rel, off, nr = stream[c]
            sl = rows(q_rel, off, nr)
            out_ref[sl, :] = x_ref[sl, :] + comm_ref[sl, :]

        def fold_y(c):
            y_rd[c].wait_recv()
            add(Y_STREAM, c)

        def fold_cw(c):
            in_wait(CW_IN, cw_r, c)
            if 5 <= c <= 6:
                ring_start(CW_OUT, cw_s, cw_r, nxt, c + 2, cw_rd)

        def fold_ccw(c):
            in_wait(CCW_IN, ccw_r, c)
            if 3 <= c <= 4:
                ring_start(CCW_OUT, ccw_s, ccw_r, prv, c + 4, ccw_rd)

        for c in range(7):
            y_rd[c].wait_recv()
            add(Y_STREAM, c)
            ring_start(CW_OUT, cw_s, cw_r, nxt, c, cw_rd)
            ring_start(CCW_OUT, ccw_s, ccw_r, prv, c, ccw_rd)
            if c == 4:
                fold_ccw(3)
            elif c == 5:
                fold_ccw(4)
            elif c == 6:
                fold_cw(5)
                fold_cw(6)
        for c in range(5):
            fold_cw(c)
        for c in range(3):
            fold_ccw(c)
        for c in range(5, 7):
            fold_ccw(c)
        for c in range(7, 9):
            fold_y(c)
        for c in range(7, 9):
            fold_cw(c)
            fold_ccw(c)

        for r in y_rd:
            r.wait_send()
        for r in cw_rd:
            r.wait_send()
        for r in ccw_rd:
            r.wait_send()

        @functools.partial(
            pl.run_scoped, second_barrier=pltpu.SemaphoreType.REGULAR
        )
        def _(second_barrier):
            for p in (peer_y, nxt, prv):
                pl.semaphore_signal(second_barrier, inc=1, device_id=p,
                                    device_id_type=MESH)
            pl.semaphore_wait(second_barrier, 3)

    nc = len(Y_STREAM)
    nr_ = len(CW_OUT)
    return pl.pallas_call(
        body,
        out_shape=jax.ShapeDtypeStruct((m, n), x.dtype),
        in_specs=[pl.BlockSpec(memory_space=pltpu.VMEM)],
        out_specs=pl.BlockSpec(memory_space=pltpu.VMEM),
        scratch_shapes=[
            pltpu.VMEM((m, n), x.dtype),
            pltpu.SemaphoreType.DMA((nc,)),
            pltpu.SemaphoreType.DMA((nc,)),
            pltpu.SemaphoreType.DMA((nr_,)),
            pltpu.SemaphoreType.DMA((nr_,)),
            pltpu.SemaphoreType.DMA((nr_,)),
            pltpu.SemaphoreType.DMA((nr_,)),
        ],
        compiler_params=pltpu.CompilerParams(collective_id=0),
    )(x)


# device time: 80999 ns/iter; 1.0769x vs baseline; 1.0769x over previous
import functools

import jax
import jax.numpy as jnp
from jax import lax
from jax.experimental import pallas as pl
from jax.experimental.pallas import tpu as pltpu

QROWS = 1024

S0, S1, S2 = 384, 320, 320
PIECE_OFF = (0, S0, S0 + S1)
CHUNK_ROWS = ((64, 112, 104, 104), (104, 104, 112), (104, 104, 112))


def _chunks(pieces):
    out = []
    for q_rel, j in pieces:
        off = PIECE_OFF[j]
        for r in CHUNK_ROWS[j]:
            out.append((q_rel, off, r))
            off += r
    return out


Y_STREAM = _chunks([(0, 0), (0, 1), (0, 2), (2, 0)])
CW_OUT = _chunks([(0, 0), (0, 1), (0, 2), (3, 2)])
CCW_OUT = _chunks([(0, 0), (0, 1), (0, 2), (1, 1)])
CW_IN = _chunks([(3, 0), (3, 1), (3, 2), (2, 2)])
CCW_IN = _chunks([(1, 0), (1, 1), (1, 2), (2, 1)])

MESH = pltpu.DeviceIdType.MESH


def kernel(x):
    m, n = x.shape

    def body(x_ref, out_ref, comm_ref, xv_ref,
             y_s, y_r, cw_s, cw_r, ccw_s, ccw_r, xin_sem, odma_sem):
        my_x = lax.axis_index("x")
        my_y = lax.axis_index("y")
        my_z = lax.axis_index("z")
        k = 2 * my_x + (my_x ^ my_z)
        even = (k % 2) == 0
        nxt = (jnp.where(even, my_x, 1 - my_x), my_y,
               jnp.where(even, 1 - my_z, my_z))
        prv = (jnp.where(even, 1 - my_x, my_x), my_y,
               jnp.where(even, my_z, 1 - my_z))
        peer_y = (my_x, 1 - my_y, my_z)

        def rows(q_rel, off, nrows):
            return pl.ds(((k + q_rel) % 4) * QROWS + off, nrows)

        barrier_sem = pltpu.get_barrier_semaphore()
        for p in (peer_y, nxt, prv):
            pl.semaphore_signal(barrier_sem, inc=1, device_id=p,
                                device_id_type=MESH)
        pl.semaphore_wait(barrier_sem, 3)

        y_rd = []
        for c, (q_rel, off, nr) in enumerate(Y_STREAM):
            sl = rows(q_rel, off, nr)
            r = pltpu.make_async_remote_copy(
                src_ref=x_ref.at[sl, :], dst_ref=comm_ref.at[sl, :],
                send_sem=y_s.at[c], recv_sem=y_r.at[c],
                device_id=peer_y, device_id_type=MESH)
            r.start()
            y_rd.append(r)

        xin = []
        for c, (q_rel, off, nr) in enumerate(Y_STREAM):
            sl = rows(q_rel, off, nr)
            cp = pltpu.make_async_copy(x_ref.at[sl, :], xv_ref.at[sl, :],
                                       xin_sem.at[c])
            cp.start()
            xin.append(cp)

        cw_rd = [None] * len(CW_OUT)
        ccw_rd = [None] * len(CCW_OUT)
        odma = [None] * len(Y_STREAM)

        def ring_start(stream, sems_s, sems_r, target, c, lst, src_ref):
            q_rel, off, nr = stream[c]
            sl = rows(q_rel, off, nr)
            r = pltpu.make_async_remote_copy(
                src_ref=src_ref.at[sl, :], dst_ref=out_ref.at[sl, :],
                send_sem=sems_s.at[c], recv_sem=sems_r.at[c],
                device_id=target, device_id_type=MESH)
            r.start()
            lst[c] = r

        def in_wait(stream, sems_r, c):
            q_rel, off, nr = stream[c]
            sl = rows(q_rel, off, nr)
            r = pltpu.make_async_remote_copy(
                src_ref=out_ref.at[sl, :], dst_ref=out_ref.at[sl, :],
                send_sem=y_s.at[0], recv_sem=sems_r.at[c],
                device_id=peer_y, device_id_type=MESH)
            r.wait_recv()

        def fold_y(c):
            q_rel, off, nr = Y_STREAM[c]
            sl = rows(q_rel, off, nr)
            y_rd[c].wait_recv()
            xin[c].wait()
            xv_ref[sl, :] = xv_ref[sl, :] + comm_ref[sl, :]
            cp = pltpu.make_async_copy(xv_ref.at[sl, :], out_ref.at[sl, :],
                                       odma_sem.at[c])
            cp.start()
            odma[c] = cp

        def fold_cw(c):
            in_wait(CW_IN, cw_r, c)
            if 7 <= c <= 9:
                ring_start(CW_OUT, cw_s, cw_r, nxt, c + 3, cw_rd, out_ref)

        def fold_ccw(c):
            in_wait(CCW_IN, ccw_r, c)
            if 4 <= c <= 6:
                ring_start(CCW_OUT, ccw_s, ccw_r, prv, c + 6, ccw_rd,
                           out_ref)

        for c in range(10):
            fold_y(c)
            ring_start(CW_OUT, cw_s, cw_r, nxt, c, cw_rd, xv_ref)
            ring_start(CCW_OUT, ccw_s, ccw_r, prv, c, ccw_rd, xv_ref)
            if c == 6:
                fold_ccw(4)
            elif c == 7:
                fold_ccw(5)
            elif c == 8:
                fold_ccw(6)
                fold_cw(7)
            elif c == 9:
                fold_cw(8)
                fold_cw(9)
        for c in range(4):
            fold_cw(c)
            fold_ccw(c)
        for c in range(4, 7):
            fold_cw(c)
        for c in range(7, 10):
            fold_ccw(c)
        for c in range(10, 14):
            fold_y(c)
        for c in range(10, 13):
            fold_cw(c)
            fold_ccw(c)

        for r in y_rd:
            r.wait_send()
        for r in cw_rd:
            r.wait_send()
        for r in ccw_rd:
            r.wait_send()
        for cp in odma:
            cp.wait()

        @functools.partial(
            pl.run_scoped, second_barrier=pltpu.SemaphoreType.REGULAR
        )
        def _(second_barrier):
            for p in (peer_y, nxt, prv):
                pl.semaphore_signal(second_barrier, inc=1, device_id=p,
                                    device_id_type=MESH)
            pl.semaphore_wait(second_barrier, 3)

    nc = len(Y_STREAM)
    nr_ = len(CW_OUT)
    return pl.pallas_call(
        body,
        out_shape=jax.ShapeDtypeStruct((m, n), x.dtype),
        in_specs=[pl.BlockSpec(memory_space=pl.ANY)],
        out_specs=pl.BlockSpec(memory_space=pl.ANY),
        scratch_shapes=[
            pltpu.VMEM((m, n), x.dtype),
            pltpu.VMEM((m, n), x.dtype),
            pltpu.SemaphoreType.DMA((nc,)),
            pltpu.SemaphoreType.DMA((nc,)),
            pltpu.SemaphoreType.DMA((nr_,)),
            pltpu.SemaphoreType.DMA((nr_,)),
            pltpu.SemaphoreType.DMA((nr_,)),
            pltpu.SemaphoreType.DMA((nr_,)),
            pltpu.SemaphoreType.DMA((nc,)),
            pltpu.SemaphoreType.DMA((nc,)),
        ],
        compiler_params=pltpu.CompilerParams(collective_id=0),
    )(x)


# device time: 80725 ns/iter; 1.0806x vs baseline; 1.0034x over previous
import functools

import jax
import jax.numpy as jnp
from jax import lax
from jax.experimental import pallas as pl
from jax.experimental.pallas import tpu as pltpu

QROWS = 1024

S0, S1, S2 = 400, 312, 312
PIECE_OFF = (0, S0, S0 + S1)
CHUNK_ROWS = ((32, 120, 128, 120), (104, 104, 104), (104, 104, 104))


def _chunks(pieces):
    out = []
    for q_rel, j in pieces:
        off = PIECE_OFF[j]
        for r in CHUNK_ROWS[j]:
            out.append((q_rel, off, r))
            off += r
    return out


Y_STREAM = _chunks([(0, 0), (0, 1), (0, 2), (2, 0)])
CW_OUT = _chunks([(0, 0), (0, 1), (0, 2), (3, 2)])
CCW_OUT = _chunks([(0, 0), (0, 1), (0, 2), (1, 1)])
CW_IN = _chunks([(3, 0), (3, 1), (3, 2), (2, 2)])
CCW_IN = _chunks([(1, 0), (1, 1), (1, 2), (2, 1)])

MESH = pltpu.DeviceIdType.MESH


def kernel(x):
    m, n = x.shape

    def body(x_ref, out_ref, comm_ref, xv_ref,
             y_s, y_r, cw_s, cw_r, ccw_s, ccw_r, xin_sem, odma_sem):
        my_x = lax.axis_index("x")
        my_y = lax.axis_index("y")
        my_z = lax.axis_index("z")
        k = 2 * my_x + (my_x ^ my_z)
        even = (k % 2) == 0
        nxt = (jnp.where(even, my_x, 1 - my_x), my_y,
               jnp.where(even, 1 - my_z, my_z))
        prv = (jnp.where(even, 1 - my_x, my_x), my_y,
               jnp.where(even, my_z, 1 - my_z))
        peer_y = (my_x, 1 - my_y, my_z)

        def rows(q_rel, off, nrows):
            return pl.ds(((k + q_rel) % 4) * QROWS + off, nrows)

        barrier_sem = pltpu.get_barrier_semaphore()
        for p in (peer_y, nxt, prv):
            pl.semaphore_signal(barrier_sem, inc=1, device_id=p,
                                device_id_type=MESH)
        pl.semaphore_wait(barrier_sem, 3)

        y_rd = []
        for c, (q_rel, off, nr) in enumerate(Y_STREAM):
            sl = rows(q_rel, off, nr)
            r = pltpu.make_async_remote_copy(
                src_ref=x_ref.at[sl, :], dst_ref=comm_ref.at[sl, :],
                send_sem=y_s.at[c], recv_sem=y_r.at[c],
                device_id=peer_y, device_id_type=MESH)
            r.start()
            y_rd.append(r)

        xin = []
        for c, (q_rel, off, nr) in enumerate(Y_STREAM):
            sl = rows(q_rel, off, nr)
            cp = pltpu.make_async_copy(x_ref.at[sl, :], xv_ref.at[sl, :],
                                       xin_sem.at[c])
            cp.start()
            xin.append(cp)

        cw_rd = [None] * len(CW_OUT)
        ccw_rd = [None] * len(CCW_OUT)
        odma = [None] * len(Y_STREAM)

        def ring_start(stream, sems_s, sems_r, target, c, lst, src_ref):
            q_rel, off, nr = stream[c]
            sl = rows(q_rel, off, nr)
            r = pltpu.make_async_remote_copy(
                src_ref=src_ref.at[sl, :], dst_ref=out_ref.at[sl, :],
                send_sem=sems_s.at[c], recv_sem=sems_r.at[c],
                device_id=target, device_id_type=MESH)
            r.start()
            lst[c] = r

        def in_wait(stream, sems_r, c):
            q_rel, off, nr = stream[c]
            sl = rows(q_rel, off, nr)
            r = pltpu.make_async_remote_copy(
                src_ref=out_ref.at[sl, :], dst_ref=out_ref.at[sl, :],
                send_sem=y_s.at[0], recv_sem=sems_r.at[c],
                device_id=peer_y, device_id_type=MESH)
            r.wait_recv()

        def fold_y(c):
            q_rel, off, nr = Y_STREAM[c]
            sl = rows(q_rel, off, nr)
            y_rd[c].wait_recv()
            xin[c].wait()
            xv_ref[sl, :] = xv_ref[sl, :] + comm_ref[sl, :]
            cp = pltpu.make_async_copy(xv_ref.at[sl, :], out_ref.at[sl, :],
                                       odma_sem.at[c])
            cp.start()
            odma[c] = cp

        def fold_cw(c):
            in_wait(CW_IN, cw_r, c)
            if 7 <= c <= 9:
                ring_start(CW_OUT, cw_s, cw_r, nxt, c + 3, cw_rd, out_ref)

        def fold_ccw(c):
            in_wait(CCW_IN, ccw_r, c)
            if 4 <= c <= 6:
                ring_start(CCW_OUT, ccw_s, ccw_r, prv, c + 6, ccw_rd,
                           out_ref)

        for c in range(10):
            fold_y(c)
            ring_start(CW_OUT, cw_s, cw_r, nxt, c, cw_rd, xv_ref)
            ring_start(CCW_OUT, ccw_s, ccw_r, prv, c, ccw_rd, xv_ref)
            if c == 6:
                fold_ccw(4)
            elif c == 7:
                fold_ccw(5)
            elif c == 8:
                fold_ccw(6)
                fold_cw(7)
            elif c == 9:
                fold_cw(8)
                fold_cw(9)
        for c in range(4):
            fold_cw(c)
            fold_ccw(c)
        for c in range(4, 7):
            fold_cw(c)
        for c in range(7, 10):
            fold_ccw(c)
        for c in range(10, 14):
            fold_y(c)
        for c in range(10, 13):
            fold_cw(c)
            fold_ccw(c)

        for r in y_rd:
            r.wait_send()
        for r in cw_rd:
            r.wait_send()
        for r in ccw_rd:
            r.wait_send()
        for cp in odma:
            cp.wait()

        @functools.partial(
            pl.run_scoped, second_barrier=pltpu.SemaphoreType.REGULAR
        )
        def _(second_barrier):
            for p in (peer_y, nxt, prv):
                pl.semaphore_signal(second_barrier, inc=1, device_id=p,
                                    device_id_type=MESH)
            pl.semaphore_wait(second_barrier, 3)

    nc = len(Y_STREAM)
    nr_ = len(CW_OUT)
    return pl.pallas_call(
        body,
        out_shape=jax.ShapeDtypeStruct((m, n), x.dtype),
        in_specs=[pl.BlockSpec(memory_space=pl.ANY)],
        out_specs=pl.BlockSpec(memory_space=pl.ANY),
        scratch_shapes=[
            pltpu.VMEM((m, n), x.dtype),
            pltpu.VMEM((m, n), x.dtype),
            pltpu.SemaphoreType.DMA((nc,)),
            pltpu.SemaphoreType.DMA((nc,)),
            pltpu.SemaphoreType.DMA((nr_,)),
            pltpu.SemaphoreType.DMA((nr_,)),
            pltpu.SemaphoreType.DMA((nr_,)),
            pltpu.SemaphoreType.DMA((nr_,)),
            pltpu.SemaphoreType.DMA((nc,)),
            pltpu.SemaphoreType.DMA((nc,)),
        ],
        compiler_params=pltpu.CompilerParams(collective_id=0),
    )(x)


# device time: 80593 ns/iter; 1.0824x vs baseline; 1.0016x over previous
import functools

import jax
import jax.numpy as jnp
from jax import lax
from jax.experimental import pallas as pl
from jax.experimental.pallas import tpu as pltpu

QROWS = 1024

S0, S1, S2 = 400, 312, 312
PIECE_OFF = (0, S0, S0 + S1)
CHUNK_ROWS = ((32, 120, 128, 120), (104, 104, 104), (104, 104, 104))


def _chunks(pieces):
    out = []
    for q_rel, j in pieces:
        off = PIECE_OFF[j]
        for r in CHUNK_ROWS[j]:
            out.append((q_rel, off, r))
            off += r
    return out


Y_STREAM = _chunks([(0, 0), (0, 1), (0, 2), (2, 0)])
CW_OUT = _chunks([(0, 0), (0, 1), (0, 2), (3, 2)])
CCW_OUT = _chunks([(0, 0), (0, 1), (0, 2), (1, 1)])
CW_IN = _chunks([(3, 0), (3, 1), (3, 2), (2, 2)])
CCW_IN = _chunks([(1, 0), (1, 1), (1, 2), (2, 1)])

MESH = pltpu.DeviceIdType.MESH


def kernel(x):
    m, n = x.shape

    def body(x_ref, out_ref, comm_ref, xv_ref,
             y_s, y_r, cw_s, cw_r, ccw_s, ccw_r, xin_sem, odma_sem):
        my_x = lax.axis_index("x")
        my_y = lax.axis_index("y")
        my_z = lax.axis_index("z")
        k = 2 * my_x + (my_x ^ my_z)
        even = (k % 2) == 0
        nxt = (jnp.where(even, my_x, 1 - my_x), my_y,
               jnp.where(even, 1 - my_z, my_z))
        prv = (jnp.where(even, 1 - my_x, my_x), my_y,
               jnp.where(even, my_z, 1 - my_z))
        peer_y = (my_x, 1 - my_y, my_z)

        def rows(q_rel, off, nrows):
            return pl.ds(((k + q_rel) % 4) * QROWS + off, nrows)

        xin = []
        for c, (q_rel, off, nr) in enumerate(Y_STREAM):
            sl = rows(q_rel, off, nr)
            cp = pltpu.make_async_copy(x_ref.at[sl, :], xv_ref.at[sl, :],
                                       xin_sem.at[c])
            cp.start()
            xin.append(cp)

        barrier_sem = pltpu.get_barrier_semaphore()
        for p in (peer_y, nxt, prv):
            pl.semaphore_signal(barrier_sem, inc=1, device_id=p,
                                device_id_type=MESH)
        pl.semaphore_wait(barrier_sem, 3)

        y_rd = []
        for c, (q_rel, off, nr) in enumerate(Y_STREAM):
            sl = rows(q_rel, off, nr)
            xin[c].wait()
            r = pltpu.make_async_remote_copy(
                src_ref=xv_ref.at[sl, :], dst_ref=comm_ref.at[sl, :],
                send_sem=y_s.at[c], recv_sem=y_r.at[c],
                device_id=peer_y, device_id_type=MESH)
            r.start()
            y_rd.append(r)

        cw_rd = [None] * len(CW_OUT)
        ccw_rd = [None] * len(CCW_OUT)
        odma = [None] * len(Y_STREAM)

        def ring_start(stream, sems_s, sems_r, target, c, lst, src_ref):
            q_rel, off, nr = stream[c]
            sl = rows(q_rel, off, nr)
            r = pltpu.make_async_remote_copy(
                src_ref=src_ref.at[sl, :], dst_ref=out_ref.at[sl, :],
                send_sem=sems_s.at[c], recv_sem=sems_r.at[c],
                device_id=target, device_id_type=MESH)
            r.start()
            lst[c] = r

        def in_wait(stream, sems_r, c):
            q_rel, off, nr = stream[c]
            sl = rows(q_rel, off, nr)
            r = pltpu.make_async_remote_copy(
                src_ref=out_ref.at[sl, :], dst_ref=out_ref.at[sl, :],
                send_sem=y_s.at[0], recv_sem=sems_r.at[c],
                device_id=peer_y, device_id_type=MESH)
            r.wait_recv()

        def fold_y(c):
            q_rel, off, nr = Y_STREAM[c]
            sl = rows(q_rel, off, nr)
            y_rd[c].wait_recv()
            y_rd[c].wait_send()
            xv_ref[sl, :] = xv_ref[sl, :] + comm_ref[sl, :]
            cp = pltpu.make_async_copy(xv_ref.at[sl, :], out_ref.at[sl, :],
                                       odma_sem.at[c])
            cp.start()
            odma[c] = cp

        def fold_cw(c):
            in_wait(CW_IN, cw_r, c)
            if 7 <= c <= 9:
                ring_start(CW_OUT, cw_s, cw_r, nxt, c + 3, cw_rd, out_ref)

        def fold_ccw(c):
            in_wait(CCW_IN, ccw_r, c)
            if 4 <= c <= 6:
                ring_start(CCW_OUT, ccw_s, ccw_r, prv, c + 6, ccw_rd,
                           out_ref)

        for c in range(10):
            fold_y(c)
            ring_start(CW_OUT, cw_s, cw_r, nxt, c, cw_rd, xv_ref)
            ring_start(CCW_OUT, ccw_s, ccw_r, prv, c, ccw_rd, xv_ref)
            if c == 6:
                fold_ccw(4)
            elif c == 7:
                fold_ccw(5)
            elif c == 8:
                fold_ccw(6)
                fold_cw(7)
            elif c == 9:
                fold_cw(8)
                fold_cw(9)
        for c in range(4):
            fold_cw(c)
            fold_ccw(c)
        for c in range(4, 7):
            fold_cw(c)
        for c in range(7, 10):
            fold_ccw(c)
        for c in range(10, 14):
            fold_y(c)
        for c in range(10, 13):
            fold_cw(c)
            fold_ccw(c)

        for r in cw_rd:
            r.wait_send()
        for r in ccw_rd:
            r.wait_send()
        for cp in odma:
            cp.wait()

        @functools.partial(
            pl.run_scoped, second_barrier=pltpu.SemaphoreType.REGULAR
        )
        def _(second_barrier):
            for p in (peer_y, nxt, prv):
                pl.semaphore_signal(second_barrier, inc=1, device_id=p,
                                    device_id_type=MESH)
            pl.semaphore_wait(second_barrier, 3)

    nc = len(Y_STREAM)
    nr_ = len(CW_OUT)
    return pl.pallas_call(
        body,
        out_shape=jax.ShapeDtypeStruct((m, n), x.dtype),
        in_specs=[pl.BlockSpec(memory_space=pl.ANY)],
        out_specs=pl.BlockSpec(memory_space=pl.ANY),
        scratch_shapes=[
            pltpu.VMEM((m, n), x.dtype),
            pltpu.VMEM((m, n), x.dtype),
            pltpu.SemaphoreType.DMA((nc,)),
            pltpu.SemaphoreType.DMA((nc,)),
            pltpu.SemaphoreType.DMA((nr_,)),
            pltpu.SemaphoreType.DMA((nr_,)),
            pltpu.SemaphoreType.DMA((nr_,)),
            pltpu.SemaphoreType.DMA((nr_,)),
            pltpu.SemaphoreType.DMA((nc,)),
            pltpu.SemaphoreType.DMA((nc,)),
        ],
        compiler_params=pltpu.CompilerParams(collective_id=0),
    )(x)


# device time: 80024 ns/iter; 1.0901x vs baseline; 1.0071x over previous
import functools

import jax
import jax.numpy as jnp
from jax import lax
from jax.experimental import pallas as pl
from jax.experimental.pallas import tpu as pltpu

QROWS = 1024

S0, S1, S2 = 400, 312, 312
PIECE_OFF = (0, S0, S0 + S1)
CHUNK_ROWS = ((32, 64, 96, 104, 104), (72, 80, 80, 80), (72, 80, 80, 80))


def _chunks(pieces):
    out = []
    for q_rel, j in pieces:
        off = PIECE_OFF[j]
        for r in CHUNK_ROWS[j]:
            out.append((q_rel, off, r))
            off += r
    return out


Y_STREAM = _chunks([(0, 0), (0, 1), (0, 2), (2, 0)])
CW_OUT = _chunks([(0, 0), (0, 1), (0, 2), (3, 2)])
CCW_OUT = _chunks([(0, 0), (0, 1), (0, 2), (1, 1)])
CW_IN = _chunks([(3, 0), (3, 1), (3, 2), (2, 2)])
CCW_IN = _chunks([(1, 0), (1, 1), (1, 2), (2, 1)])

N0, N1, N2 = (len(c) for c in CHUNK_ROWS)
NQ = N0 + N1 + N2
CW_FWD_LO = N0 + N1
CCW_FWD_LO = N0

MESH = pltpu.DeviceIdType.MESH


def kernel(x):
    m, n = x.shape

    def body(x_ref, out_ref, comm_ref, xv_ref,
             y_s, y_r, cw_s, cw_r, ccw_s, ccw_r, xin_sem, odma_sem):
        my_x = lax.axis_index("x")
        my_y = lax.axis_index("y")
        my_z = lax.axis_index("z")
        k = 2 * my_x + (my_x ^ my_z)
        even = (k % 2) == 0
        nxt = (jnp.where(even, my_x, 1 - my_x), my_y,
               jnp.where(even, 1 - my_z, my_z))
        prv = (jnp.where(even, 1 - my_x, my_x), my_y,
               jnp.where(even, my_z, 1 - my_z))
        peer_y = (my_x, 1 - my_y, my_z)

        def rows(q_rel, off, nrows):
            return pl.ds(((k + q_rel) % 4) * QROWS + off, nrows)

        xin = []
        for c, (q_rel, off, nr) in enumerate(Y_STREAM):
            sl = rows(q_rel, off, nr)
            cp = pltpu.make_async_copy(x_ref.at[sl, :], xv_ref.at[sl, :],
                                       xin_sem.at[c])
            cp.start()
            xin.append(cp)

        barrier_sem = pltpu.get_barrier_semaphore()
        for p in (peer_y, nxt, prv):
            pl.semaphore_signal(barrier_sem, inc=1, device_id=p,
                                device_id_type=MESH)
        pl.semaphore_wait(barrier_sem, 3)

        y_rd = []
        for c, (q_rel, off, nr) in enumerate(Y_STREAM):
            sl = rows(q_rel, off, nr)
            xin[c].wait()
            r = pltpu.make_async_remote_copy(
                src_ref=xv_ref.at[sl, :], dst_ref=comm_ref.at[sl, :],
                send_sem=y_s.at[c], recv_sem=y_r.at[c],
                device_id=peer_y, device_id_type=MESH)
            r.start()
            y_rd.append(r)

        cw_rd = [None] * len(CW_OUT)
        ccw_rd = [None] * len(CCW_OUT)
        odma = [None] * len(Y_STREAM)

        def ring_start(stream, sems_s, sems_r, target, c, lst, src_ref):
            q_rel, off, nr = stream[c]
            sl = rows(q_rel, off, nr)
            r = pltpu.make_async_remote_copy(
                src_ref=src_ref.at[sl, :], dst_ref=out_ref.at[sl, :],
                send_sem=sems_s.at[c], recv_sem=sems_r.at[c],
                device_id=target, device_id_type=MESH)
            r.start()
            lst[c] = r

        def in_wait(stream, sems_r, c):
            q_rel, off, nr = stream[c]
            sl = rows(q_rel, off, nr)
            r = pltpu.make_async_remote_copy(
                src_ref=out_ref.at[sl, :], dst_ref=out_ref.at[sl, :],
                send_sem=y_s.at[0], recv_sem=sems_r.at[c],
                device_id=peer_y, device_id_type=MESH)
            r.wait_recv()

        def fold_y(c):
            q_rel, off, nr = Y_STREAM[c]
            sl = rows(q_rel, off, nr)
            y_rd[c].wait_recv()
            y_rd[c].wait_send()
            xv_ref[sl, :] = xv_ref[sl, :] + comm_ref[sl, :]
            cp = pltpu.make_async_copy(xv_ref.at[sl, :], out_ref.at[sl, :],
                                       odma_sem.at[c])
            cp.start()
            odma[c] = cp

        def fold_cw(c):
            in_wait(CW_IN, cw_r, c)
            if CW_FWD_LO <= c < CW_FWD_LO + N2:
                ring_start(CW_OUT, cw_s, cw_r, nxt, NQ + (c - CW_FWD_LO),
                           cw_rd, out_ref)

        def fold_ccw(c):
            in_wait(CCW_IN, ccw_r, c)
            if CCW_FWD_LO <= c < CCW_FWD_LO + N1:
                ring_start(CCW_OUT, ccw_s, ccw_r, prv,
                           NQ + (c - CCW_FWD_LO), ccw_rd, out_ref)

        for c in range(NQ):
            fold_y(c)
            ring_start(CW_OUT, cw_s, cw_r, nxt, c, cw_rd, xv_ref)
            ring_start(CCW_OUT, ccw_s, ccw_r, prv, c, ccw_rd, xv_ref)
            if c == 8:
                fold_ccw(5)
            elif c == 9:
                fold_ccw(6)
            elif c == 10:
                fold_ccw(7)
            elif c == 11:
                fold_ccw(8)
                fold_cw(9)
            elif c == 12:
                fold_cw(10)
                fold_cw(11)
                fold_cw(12)
        for c in range(9):
            fold_cw(c)
        for c in range(5):
            fold_ccw(c)
        for c in range(9, 13):
            fold_ccw(c)
        for c in range(NQ, len(Y_STREAM)):
            fold_y(c)
        for c in range(13, 17):
            fold_cw(c)
            fold_ccw(c)

        for r in cw_rd:
            r.wait_send()
        for r in ccw_rd:
            r.wait_send()
        for cp in odma:
            cp.wait()

        @functools.partial(
            pl.run_scoped, second_barrier=pltpu.SemaphoreType.REGULAR
        )
        def _(second_barrier):
            for p in (peer_y, nxt, prv):
                pl.semaphore_signal(second_barrier, inc=1, device_id=p,
                                    device_id_type=MESH)
            pl.semaphore_wait(second_barrier, 3)

    nc = len(Y_STREAM)
    nr_ = len(CW_OUT)
    return pl.pallas_call(
        body,
        out_shape=jax.ShapeDtypeStruct((m, n), x.dtype),
        in_specs=[pl.BlockSpec(memory_space=pl.ANY)],
        out_specs=pl.BlockSpec(memory_space=pl.ANY),
        scratch_shapes=[
            pltpu.VMEM((m, n), x.dtype),
            pltpu.VMEM((m, n), x.dtype),
            pltpu.SemaphoreType.DMA((nc,)),
            pltpu.SemaphoreType.DMA((nc,)),
            pltpu.SemaphoreType.DMA((nr_,)),
            pltpu.SemaphoreType.DMA((nr_,)),
            pltpu.SemaphoreType.DMA((nr_,)),
            pltpu.SemaphoreType.DMA((nr_,)),
            pltpu.SemaphoreType.DMA((nc,)),
            pltpu.SemaphoreType.DMA((nc,)),
        ],
        compiler_params=pltpu.CompilerParams(collective_id=0),
    )(x)
